# Initial kernel scaffold; baseline (speedup 1.0000x reference)
import functools
import jax
import jax.numpy as jnp
from jax import lax
from jax.experimental import pallas as pl
from jax.experimental.pallas import tpu as pltpu
from jax.experimental.pallas import tpu_sc as plsc

_NUM_R = 230
_D = 128
_P = 16384
_L = 8
_B = 1024
_K = 32
_TT_PAD = 4096
_REL_PAD = 256
_NW = 32
_GRU_BLK = 1024
_SC_BLK = 128

_PREC = lax.Precision.HIGHEST


def _normalize_rows(x):
    n = jnp.sqrt(jnp.sum(x * x, axis=1, keepdims=True))
    return x / jnp.maximum(n, 1e-12)


def _prep_body(tt_ref, wlt_ref, bl_ref, rel_ref, ttlin_ref, reln_ref):
    ttlin_ref[...] = (
        jnp.dot(tt_ref[...], wlt_ref[...],
                preferred_element_type=jnp.float32, precision=_PREC)
        + bl_ref[...]
    )
    reln_ref[...] = _normalize_rows(rel_ref[...])


def _prep(tt_pad, w_lin_t, b_lin2, rel_pad):
    return pl.pallas_call(
        _prep_body,
        out_shape=(
            jax.ShapeDtypeStruct((_TT_PAD, _D), jnp.float32),
            jax.ShapeDtypeStruct((_REL_PAD, _D), jnp.float32),
        ),
    )(tt_pad, w_lin_t, b_lin2, rel_pad)


def _make_sc_gather(n_rows, chunk):
    n_per = n_rows // _NW
    n_ch = n_per // chunk
    assert n_per % chunk == 0 and chunk <= 128 and chunk % 8 == 0

    mesh = plsc.VectorSubcoreMesh(core_axis_name="c", subcore_axis_name="s")

    @functools.partial(
        pl.kernel,
        out_type=jax.ShapeDtypeStruct((n_rows, _D), jnp.float32),
        mesh=mesh,
        scratch_types=[
            pltpu.VMEM((n_ch, chunk), jnp.int32),
            pltpu.VMEM((chunk, _D), jnp.float32),
            pltpu.SemaphoreType.DMA,
        ],
    )
    def gather_k(table_hbm, idx_hbm, out_hbm, idx_v, rows_v, sem):
        wid = lax.axis_index("s") * 2 + lax.axis_index("c")
        base = wid * n_per
        pltpu.sync_copy(idx_hbm.at[wid], idx_v)
        for g in range(n_ch):
            pltpu.async_copy(table_hbm.at[idx_v.at[g]], rows_v, sem).wait()
            pltpu.sync_copy(rows_v, out_hbm.at[pl.ds(base + g * chunk, chunk)])

    return gather_k


def _gru_body(paths_ref, len_ref, embt_ref, relpad_ref, wiht_ref, whht_ref,
              bih_ref, bhh_ref, out_ref):
    R = _GRU_BLK
    pth = paths_ref[0, 0, :].reshape(R * _L, 1)
    oh = (pth == lax.broadcasted_iota(jnp.int32, (1, _REL_PAD), 1))
    emb = embt_ref[...] + jnp.dot(
        oh.astype(jnp.float32), relpad_ref[...],
        preferred_element_type=jnp.float32, precision=_PREC)
    gi = jnp.dot(emb, wiht_ref[...],
                 preferred_element_type=jnp.float32, precision=_PREC)
    gi = (gi + bih_ref[...]).reshape(R, _L, 3 * _D)
    lens = len_ref[0, 0, :].reshape(R, 1)

    h = jnp.zeros((R, _D), jnp.float32)
    for t in range(_L):
        gh = jnp.dot(h, whht_ref[...],
                     preferred_element_type=jnp.float32, precision=_PREC)
        gh = gh + bhh_ref[...]
        gi_t = gi[:, t, :]
        r = jax.nn.sigmoid(gi_t[:, :_D] + gh[:, :_D])
        z = jax.nn.sigmoid(gi_t[:, _D:2 * _D] + gh[:, _D:2 * _D])
        n = jnp.tanh(gi_t[:, 2 * _D:] + r * gh[:, 2 * _D:])
        hn = (1.0 - z) * n + z * h
        h = jnp.where(t < lens, hn, h)
    out_ref[...] = _normalize_rows(h)


def _gru(paths3, len3, emb_t, rel_pad, w_ih_t, w_hh_t, b_ih2, b_hh2):
    nblk = _P // _GRU_BLK
    full = lambda i: (0, 0)
    return pl.pallas_call(
        _gru_body,
        grid=(nblk,),
        in_specs=[
            pl.BlockSpec((1, 1, _GRU_BLK * _L), lambda i: (i, 0, 0)),
            pl.BlockSpec((1, 1, _GRU_BLK), lambda i: (i, 0, 0)),
            pl.BlockSpec((_GRU_BLK * _L, _D), lambda i: (i, 0)),
            pl.BlockSpec((_REL_PAD, _D), full),
            pl.BlockSpec((_D, 3 * _D), full),
            pl.BlockSpec((_D, 3 * _D), full),
            pl.BlockSpec((1, 3 * _D), full),
            pl.BlockSpec((1, 3 * _D), full),
        ],
        out_specs=pl.BlockSpec((_GRU_BLK, _D), lambda i: (i, 0)),
        out_shape=jax.ShapeDtypeStruct((_P, _D), jnp.float32),
    )(paths3, len3, emb_t, rel_pad, w_ih_t, w_hh_t, b_ih2, b_hh2)


def _score_body(rows_ref, q_ref, out_ref):
    rows = rows_ref[...].reshape(_SC_BLK, _K, _D)
    q = q_ref[...]
    s = jnp.sum(rows * q[:, None, :], axis=2)
    m = jnp.max(s, axis=1)
    out_ref[...] = jnp.maximum(m, 0.0).reshape(1, 1, _SC_BLK)


def _score(rows, q):
    nblk = _B // _SC_BLK
    out = pl.pallas_call(
        _score_body,
        grid=(nblk,),
        in_specs=[
            pl.BlockSpec((_SC_BLK * _K, _D), lambda i: (i, 0)),
            pl.BlockSpec((_SC_BLK, _D), lambda i: (i, 0)),
        ],
        out_specs=pl.BlockSpec((1, 1, _SC_BLK), lambda i: (i, 0, 0)),
        out_shape=jax.ShapeDtypeStruct((nblk, 1, _SC_BLK), jnp.float32),
    )(rows, q)
    return out.reshape(_B)


_gather_time = _make_sc_gather(_P * _L, 128)
_gather_path = _make_sc_gather(_B * (_K + 1), 96)
_gather_rel = _make_sc_gather(2 * _B, 64)


def kernel(path_index, batch_relation, paths, paths_time, lengths, path_r,
           path_neg_index, batch_his_r, relation_embeddings, time_table,
           W_lin, b_lin, W_ih, W_hh, b_ih, b_hh):
    f32 = jnp.float32
    tt_pad = jnp.zeros((_TT_PAD, _D), f32).at[:time_table.shape[0]].set(time_table)
    rel_pad = jnp.zeros((_REL_PAD, _D), f32).at[:_NUM_R].set(relation_embeddings)
    w_lin_t = W_lin.T
    w_ih_t = W_ih.T
    w_hh_t = W_hh.T
    b_lin2 = b_lin.reshape(1, _D)
    b_ih2 = b_ih.reshape(1, 3 * _D)
    b_hh2 = b_hh.reshape(1, 3 * _D)

    tt_lin, rel_n = _prep(tt_pad, w_lin_t, b_lin2, rel_pad)

    idx_t = paths_time.reshape(_NW, (_P * _L) // (_NW * 128), 128)
    emb_t = _gather_time(tt_lin, idx_t)

    paths3 = paths.reshape(_P // _GRU_BLK, 1, _GRU_BLK * _L)
    len3 = lengths.reshape(_P // _GRU_BLK, 1, _GRU_BLK).astype(jnp.int32)
    hidden_n = _gru(paths3, len3, emb_t, rel_pad, w_ih_t, w_hh_t, b_ih2, b_hh2)

    path_tbl = jnp.concatenate([jnp.zeros((8, _D), f32), hidden_n], axis=0)

    idx_p = (jnp.concatenate([path_index.reshape(-1), path_neg_index]) + 7)
    idx_p = idx_p.reshape(_NW, 11, 96)
    rows_p = _gather_path(path_tbl, idx_p)

    idx_r = jnp.concatenate([path_r, batch_relation]).reshape(_NW, 1, 64)
    rows_r = _gather_rel(rel_n, idx_r)

    max_score = _score(rows_p[:_B * _K], rows_r[_B:])
    return (max_score, rows_p[_B * _K:], rows_r[:_B])


# speedup vs baseline: 1.3290x; 1.3290x over previous
import functools
import jax
import jax.numpy as jnp
from jax import lax
from jax.experimental import pallas as pl
from jax.experimental.pallas import tpu as pltpu
from jax.experimental.pallas import tpu_sc as plsc

_NUM_R = 230
_D = 128
_P = 16384
_L = 8
_B = 1024
_K = 32
_TT_PAD = 4096
_REL_PAD = 256
_NW = 32
_GRU_BLK = 1024
_SC_BLK = 128

_PREC = lax.Precision.HIGHEST


def _normalize_rows(x):
    n = jnp.sqrt(jnp.sum(x * x, axis=1, keepdims=True))
    return x / jnp.maximum(n, 1e-12)


def _prep_body(tt_ref, wlt_ref, bl_ref, rel_ref, ttlin_ref, reln_ref):
    ttlin_ref[...] = (
        jnp.dot(tt_ref[...], wlt_ref[...],
                preferred_element_type=jnp.float32, precision=_PREC)
        + bl_ref[...]
    )
    reln_ref[...] = _normalize_rows(rel_ref[...])


def _prep(tt_pad, w_lin_t, b_lin2, rel_pad):
    return pl.pallas_call(
        _prep_body,
        out_shape=(
            jax.ShapeDtypeStruct((_TT_PAD, _D), jnp.float32),
            jax.ShapeDtypeStruct((_REL_PAD, _D), jnp.float32),
        ),
    )(tt_pad, w_lin_t, b_lin2, rel_pad)


@functools.lru_cache(maxsize=None)
def _make_sc_gather(n_rows, chunk):
    n_per = n_rows // _NW
    n_ch = n_per // chunk
    assert n_per % chunk == 0 and chunk <= 128 and chunk % 8 == 0

    mesh = plsc.VectorSubcoreMesh(core_axis_name="c", subcore_axis_name="s")

    @functools.partial(
        pl.kernel,
        out_type=jax.ShapeDtypeStruct((n_rows, _D), jnp.float32),
        mesh=mesh,
        scratch_types=[
            pltpu.VMEM((n_ch, chunk), jnp.int32),
            pltpu.VMEM((chunk, _D), jnp.float32),
            pltpu.SemaphoreType.DMA,
        ],
    )
    def gather_k(table_hbm, idx_hbm, out_hbm, idx_v, rows_v, sem):
        wid = lax.axis_index("s") * 2 + lax.axis_index("c")
        base = wid * n_per
        pltpu.sync_copy(idx_hbm.at[wid], idx_v)
        for g in range(n_ch):
            pltpu.async_copy(table_hbm.at[idx_v.at[g]], rows_v, sem).wait()
            pltpu.sync_copy(rows_v, out_hbm.at[pl.ds(base + g * chunk, chunk)])

    return gather_k


def _gru_body(paths_ref, len_ref, embt_ref, relpad_ref, wiht_ref, whht_ref,
              bih_ref, bhh_ref, out_ref):
    R = _GRU_BLK
    pth = paths_ref[0, 0, :].reshape(R * _L, 1)
    oh = (pth == lax.broadcasted_iota(jnp.int32, (1, _REL_PAD), 1))
    emb = embt_ref[...] + jnp.dot(
        oh.astype(jnp.float32), relpad_ref[...],
        preferred_element_type=jnp.float32, precision=_PREC)
    gi = jnp.dot(emb, wiht_ref[...],
                 preferred_element_type=jnp.float32, precision=_PREC)
    gi = (gi + bih_ref[...]).reshape(R, _L, 3 * _D)
    lens = len_ref[0, 0, :].reshape(R, 1)

    h = jnp.zeros((R, _D), jnp.float32)
    for t in range(_L):
        gh = jnp.dot(h, whht_ref[...],
                     preferred_element_type=jnp.float32, precision=_PREC)
        gh = gh + bhh_ref[...]
        gi_t = gi[:, t, :]
        r = jax.nn.sigmoid(gi_t[:, :_D] + gh[:, :_D])
        z = jax.nn.sigmoid(gi_t[:, _D:2 * _D] + gh[:, _D:2 * _D])
        n = jnp.tanh(gi_t[:, 2 * _D:] + r * gh[:, 2 * _D:])
        hn = (1.0 - z) * n + z * h
        h = jnp.where(t < lens, hn, h)
    out_ref[...] = _normalize_rows(h)


def _gru(paths3, len3, emb_t, rel_pad, w_ih_t, w_hh_t, b_ih2, b_hh2):
    nblk = _P // _GRU_BLK
    full = lambda i: (0, 0)
    return pl.pallas_call(
        _gru_body,
        grid=(nblk,),
        in_specs=[
            pl.BlockSpec((1, 1, _GRU_BLK * _L), lambda i: (i, 0, 0)),
            pl.BlockSpec((1, 1, _GRU_BLK), lambda i: (i, 0, 0)),
            pl.BlockSpec((_GRU_BLK * _L, _D), lambda i: (i, 0)),
            pl.BlockSpec((_REL_PAD, _D), full),
            pl.BlockSpec((_D, 3 * _D), full),
            pl.BlockSpec((_D, 3 * _D), full),
            pl.BlockSpec((1, 3 * _D), full),
            pl.BlockSpec((1, 3 * _D), full),
        ],
        out_specs=pl.BlockSpec((_GRU_BLK, _D), lambda i: (i, 0)),
        out_shape=jax.ShapeDtypeStruct((_P, _D), jnp.float32),
    )(paths3, len3, emb_t, rel_pad, w_ih_t, w_hh_t, b_ih2, b_hh2)


def _score_body(rows_ref, q_ref, out_ref):
    rows = rows_ref[...].reshape(_SC_BLK, _K, _D)
    q = q_ref[...]
    s = jnp.sum(rows * q[:, None, :], axis=2)
    m = jnp.max(s, axis=1)
    out_ref[...] = jnp.maximum(m, 0.0).reshape(1, 1, _SC_BLK)


def _score(rows, q):
    nblk = _B // _SC_BLK
    out = pl.pallas_call(
        _score_body,
        grid=(nblk,),
        in_specs=[
            pl.BlockSpec((_SC_BLK * _K, _D), lambda i: (i, 0)),
            pl.BlockSpec((_SC_BLK, _D), lambda i: (i, 0)),
        ],
        out_specs=pl.BlockSpec((1, 1, _SC_BLK), lambda i: (i, 0, 0)),
        out_shape=jax.ShapeDtypeStruct((nblk, 1, _SC_BLK), jnp.float32),
    )(rows, q)
    return out.reshape(_B)


def _gather(table, idx, n_rows, chunk):
    return _make_sc_gather(n_rows, chunk)(table, idx)


def kernel(path_index, batch_relation, paths, paths_time, lengths, path_r,
           path_neg_index, batch_his_r, relation_embeddings, time_table,
           W_lin, b_lin, W_ih, W_hh, b_ih, b_hh):
    f32 = jnp.float32
    tt_pad = jnp.zeros((_TT_PAD, _D), f32).at[:time_table.shape[0]].set(time_table)
    rel_pad = jnp.zeros((_REL_PAD, _D), f32).at[:_NUM_R].set(relation_embeddings)
    w_lin_t = W_lin.T
    w_ih_t = W_ih.T
    w_hh_t = W_hh.T
    b_lin2 = b_lin.reshape(1, _D)
    b_ih2 = b_ih.reshape(1, 3 * _D)
    b_hh2 = b_hh.reshape(1, 3 * _D)

    tt_lin, rel_n = _prep(tt_pad, w_lin_t, b_lin2, rel_pad)

    idx_t = paths_time.reshape(_NW, (_P * _L) // (_NW * 128), 128)
    emb_t = _gather(tt_lin, idx_t, _P * _L, 128)

    paths3 = paths.reshape(_P // _GRU_BLK, 1, _GRU_BLK * _L)
    len3 = lengths.reshape(_P // _GRU_BLK, 1, _GRU_BLK).astype(jnp.int32)
    hidden_n = _gru(paths3, len3, emb_t, rel_pad, w_ih_t, w_hh_t, b_ih2, b_hh2)

    path_tbl = jnp.concatenate([jnp.zeros((8, _D), f32), hidden_n], axis=0)

    idx_p = (jnp.concatenate([path_index.reshape(-1), path_neg_index]) + 7)
    idx_p = idx_p.reshape(_NW, 11, 96)
    rows_p = _gather(path_tbl, idx_p, _B * (_K + 1), 96)

    idx_r = jnp.concatenate([path_r, batch_relation]).reshape(_NW, 1, 64)
    rows_r = _gather(rel_n, idx_r, 2 * _B, 64)

    max_score = _score(rows_p[:_B * _K], rows_r[_B:])
    return (max_score, rows_p[_B * _K:], rows_r[:_B])


# speedup vs baseline: 2.4587x; 1.8500x over previous
import functools
import jax
import jax.numpy as jnp
from jax import lax
from jax.experimental import pallas as pl
from jax.experimental.pallas import tpu as pltpu
from jax.experimental.pallas import tpu_sc as plsc

_NUM_R = 230
_D = 128
_P = 16384
_L = 8
_B = 1024
_K = 32
_TT_PAD = 4096
_REL_PAD = 256
_NW = 32
_GRU_BLK = 1024
_SC_BLK = 128

_PREC = lax.Precision.DEFAULT


def _normalize_rows(x):
    n = jnp.sqrt(jnp.sum(x * x, axis=1, keepdims=True))
    return x / jnp.maximum(n, 1e-12)


def _prep_body(tt_ref, wlt_ref, bl_ref, rel_ref, ttlin_ref, reln_ref):
    ttlin_ref[...] = (
        jnp.dot(tt_ref[...], wlt_ref[...],
                preferred_element_type=jnp.float32, precision=_PREC)
        + bl_ref[...]
    )
    reln_ref[...] = _normalize_rows(rel_ref[...])


def _prep(tt_pad, w_lin_t, b_lin2, rel_pad):
    return pl.pallas_call(
        _prep_body,
        out_shape=(
            jax.ShapeDtypeStruct((_TT_PAD, _D), jnp.float32),
            jax.ShapeDtypeStruct((_REL_PAD, _D), jnp.float32),
        ),
    )(tt_pad, w_lin_t, b_lin2, rel_pad)


@functools.lru_cache(maxsize=None)
def _make_sc_gather(n_rows, chunk):
    n_per = n_rows // _NW
    n_ch = n_per // chunk
    assert n_per % chunk == 0 and chunk <= 128 and chunk % 8 == 0

    mesh = plsc.VectorSubcoreMesh(core_axis_name="c", subcore_axis_name="s")

    @functools.partial(
        pl.kernel,
        out_type=jax.ShapeDtypeStruct((n_rows, _D), jnp.float32),
        mesh=mesh,
        scratch_types=[
            pltpu.VMEM((n_ch, chunk), jnp.int32),
            pltpu.VMEM((chunk, _D), jnp.float32),
            pltpu.SemaphoreType.DMA,
        ],
    )
    def gather_k(table_hbm, idx_hbm, out_hbm, idx_v, rows_v, sem):
        wid = lax.axis_index("s") * 2 + lax.axis_index("c")
        base = wid * n_per
        pltpu.sync_copy(idx_hbm.at[wid], idx_v)
        for g in range(n_ch):
            pltpu.async_copy(table_hbm.at[idx_v.at[g]], rows_v, sem).wait()
            pltpu.sync_copy(rows_v, out_hbm.at[pl.ds(base + g * chunk, chunk)])

    return gather_k


def _gru_body(paths_ref, len_ref, embt_ref, relpad_ref, wiht_ref, whht_ref,
              bih_ref, bhh_ref, out_ref):
    R = _GRU_BLK
    pth = paths_ref[0, 0, :].reshape(R * _L, 1)
    oh = (pth == lax.broadcasted_iota(jnp.int32, (1, _REL_PAD), 1))
    emb = embt_ref[...] + jnp.dot(
        oh.astype(jnp.float32), relpad_ref[...],
        preferred_element_type=jnp.float32, precision=_PREC)
    gi = jnp.dot(emb, wiht_ref[...],
                 preferred_element_type=jnp.float32, precision=_PREC)
    gi = (gi + bih_ref[...]).reshape(R, _L, 3 * _D)
    lens = len_ref[0, 0, :].reshape(R, 1)

    h = jnp.zeros((R, _D), jnp.float32)
    for t in range(_L):
        gh = jnp.dot(h, whht_ref[...],
                     preferred_element_type=jnp.float32, precision=_PREC)
        gh = gh + bhh_ref[...]
        gi_t = gi[:, t, :]
        r = jax.nn.sigmoid(gi_t[:, :_D] + gh[:, :_D])
        z = jax.nn.sigmoid(gi_t[:, _D:2 * _D] + gh[:, _D:2 * _D])
        n = jnp.tanh(gi_t[:, 2 * _D:] + r * gh[:, 2 * _D:])
        hn = (1.0 - z) * n + z * h
        h = jnp.where(t < lens, hn, h)
    out_ref[...] = _normalize_rows(h)


def _gru(paths3, len3, emb_t, rel_pad, w_ih_t, w_hh_t, b_ih2, b_hh2):
    nblk = _P // _GRU_BLK
    full = lambda i: (0, 0)
    return pl.pallas_call(
        _gru_body,
        grid=(nblk,),
        in_specs=[
            pl.BlockSpec((1, 1, _GRU_BLK * _L), lambda i: (i, 0, 0)),
            pl.BlockSpec((1, 1, _GRU_BLK), lambda i: (i, 0, 0)),
            pl.BlockSpec((_GRU_BLK * _L, _D), lambda i: (i, 0)),
            pl.BlockSpec((_REL_PAD, _D), full),
            pl.BlockSpec((_D, 3 * _D), full),
            pl.BlockSpec((_D, 3 * _D), full),
            pl.BlockSpec((1, 3 * _D), full),
            pl.BlockSpec((1, 3 * _D), full),
        ],
        out_specs=pl.BlockSpec((_GRU_BLK, _D), lambda i: (i, 0)),
        out_shape=jax.ShapeDtypeStruct((_P, _D), jnp.float32),
    )(paths3, len3, emb_t, rel_pad, w_ih_t, w_hh_t, b_ih2, b_hh2)


def _score_body(rows_ref, q_ref, out_ref):
    rows = rows_ref[...].reshape(_SC_BLK, _K, _D)
    q = q_ref[...]
    s = jnp.sum(rows * q[:, None, :], axis=2)
    m = jnp.max(s, axis=1)
    out_ref[...] = jnp.maximum(m, 0.0).reshape(1, 1, _SC_BLK)


def _score(rows, q):
    nblk = _B // _SC_BLK
    out = pl.pallas_call(
        _score_body,
        grid=(nblk,),
        in_specs=[
            pl.BlockSpec((_SC_BLK * _K, _D), lambda i: (i, 0)),
            pl.BlockSpec((_SC_BLK, _D), lambda i: (i, 0)),
        ],
        out_specs=pl.BlockSpec((1, 1, _SC_BLK), lambda i: (i, 0, 0)),
        out_shape=jax.ShapeDtypeStruct((nblk, 1, _SC_BLK), jnp.float32),
    )(rows, q)
    return out.reshape(_B)


def _gather(table, idx, n_rows, chunk):
    return _make_sc_gather(n_rows, chunk)(table, idx)


def kernel(path_index, batch_relation, paths, paths_time, lengths, path_r,
           path_neg_index, batch_his_r, relation_embeddings, time_table,
           W_lin, b_lin, W_ih, W_hh, b_ih, b_hh):
    f32 = jnp.float32
    tt_pad = jnp.zeros((_TT_PAD, _D), f32).at[:time_table.shape[0]].set(time_table)
    rel_pad = jnp.zeros((_REL_PAD, _D), f32).at[:_NUM_R].set(relation_embeddings)
    w_lin_t = W_lin.T
    w_ih_t = W_ih.T
    w_hh_t = W_hh.T
    b_lin2 = b_lin.reshape(1, _D)
    b_ih2 = b_ih.reshape(1, 3 * _D)
    b_hh2 = b_hh.reshape(1, 3 * _D)

    tt_lin, rel_n = _prep(tt_pad, w_lin_t, b_lin2, rel_pad)

    idx_t = paths_time.reshape(_NW, (_P * _L) // (_NW * 128), 128)
    emb_t = _gather(tt_lin, idx_t, _P * _L, 128)

    paths3 = paths.reshape(_P // _GRU_BLK, 1, _GRU_BLK * _L)
    len3 = lengths.reshape(_P // _GRU_BLK, 1, _GRU_BLK).astype(jnp.int32)
    hidden_n = _gru(paths3, len3, emb_t, rel_pad, w_ih_t, w_hh_t, b_ih2, b_hh2)

    path_tbl = jnp.concatenate([jnp.zeros((8, _D), f32), hidden_n], axis=0)

    idx_p = (jnp.concatenate([path_index.reshape(-1), path_neg_index]) + 7)
    idx_p = idx_p.reshape(_NW, 11, 96)
    rows_p = _gather(path_tbl, idx_p, _B * (_K + 1), 96)

    idx_r = jnp.concatenate([path_r, batch_relation]).reshape(_NW, 1, 64)
    rows_r = _gather(rel_n, idx_r, 2 * _B, 64)

    max_score = _score(rows_p[:_B * _K], rows_r[_B:])
    return (max_score, rows_p[_B * _K:], rows_r[:_B])


# speedup vs baseline: 5.7384x; 2.3339x over previous
import functools
import jax
import jax.numpy as jnp
from jax import lax
from jax.experimental import pallas as pl
from jax.experimental.pallas import tpu as pltpu
from jax.experimental.pallas import tpu_sc as plsc

_NUM_R = 230
_D = 128
_P = 16384
_L = 8
_B = 1024
_K = 32
_TT_PAD = 4096
_REL_PAD = 256
_NW = 32
_GRU_BLK = 1024
_SC_BLK = 128

_PREC = lax.Precision.DEFAULT


def _normalize_rows(x):
    n = jnp.sqrt(jnp.sum(x * x, axis=1, keepdims=True))
    return x / jnp.maximum(n, 1e-12)


def _prep_body(tt_ref, wlt_ref, bl_ref, rel_ref, ttlin_ref, reln_ref):
    ttlin_ref[...] = (
        jnp.dot(tt_ref[...], wlt_ref[...],
                preferred_element_type=jnp.float32, precision=_PREC)
        + bl_ref[...]
    )
    reln_ref[...] = _normalize_rows(rel_ref[...])


def _prep(tt_pad, w_lin_t, b_lin2, rel_pad):
    return pl.pallas_call(
        _prep_body,
        out_shape=(
            jax.ShapeDtypeStruct((_TT_PAD, _D), jnp.float32),
            jax.ShapeDtypeStruct((_REL_PAD, _D), jnp.float32),
        ),
    )(tt_pad, w_lin_t, b_lin2, rel_pad)


@functools.lru_cache(maxsize=None)
def _make_sc_gather(n_rows, chunk):
    n_per = n_rows // _NW
    n_ch = n_per // chunk
    assert n_per % chunk == 0 and chunk <= 128 and chunk % 8 == 0

    mesh = plsc.VectorSubcoreMesh(core_axis_name="c", subcore_axis_name="s")

    @functools.partial(
        pl.kernel,
        out_type=jax.ShapeDtypeStruct((n_rows, _D), jnp.float32),
        mesh=mesh,
        scratch_types=[
            pltpu.VMEM((n_ch, chunk), jnp.int32),
            pltpu.VMEM((2, chunk, _D), jnp.float32),
            pltpu.SemaphoreType.DMA,
            pltpu.SemaphoreType.DMA,
        ],
    )
    def gather_k(table_hbm, idx_hbm, out_hbm, idx_v, rows_v, sem_a, sem_b):
        wid = lax.axis_index("s") * 2 + lax.axis_index("c")
        base = wid * n_per
        sems = (sem_a, sem_b)
        pltpu.sync_copy(idx_hbm.at[wid], idx_v)
        cps = [None, None]
        cps[0] = pltpu.async_copy(table_hbm.at[idx_v.at[0]], rows_v.at[0], sems[0])
        for g in range(n_ch):
            if g + 1 < n_ch:
                nb = (g + 1) % 2
                cps[nb] = pltpu.async_copy(
                    table_hbm.at[idx_v.at[g + 1]], rows_v.at[nb], sems[nb])
            cb = g % 2
            cps[cb].wait()
            pltpu.sync_copy(rows_v.at[cb],
                            out_hbm.at[pl.ds(base + g * chunk, chunk)])

    return gather_k


def _gru_body(paths_ref, len_ref, embt_ref, relpad_ref, wiht_ref, whht_ref,
              bih_ref, bhh_ref, out_ref):
    R = _GRU_BLK
    pth = paths_ref[...].reshape(_L * R, 1)
    oh = (pth == lax.broadcasted_iota(jnp.int32, (1, _REL_PAD), 1))
    emb = embt_ref[...].reshape(_L * R, _D) + jnp.dot(
        oh.astype(jnp.float32), relpad_ref[...],
        preferred_element_type=jnp.float32, precision=_PREC)
    gi = jnp.dot(emb, wiht_ref[...],
                 preferred_element_type=jnp.float32, precision=_PREC)
    gi = (gi + bih_ref[...]).reshape(_L, R, 3 * _D)
    lens = len_ref[0, 0, :].reshape(R, 1)

    h = jnp.zeros((R, _D), jnp.float32)
    for t in range(_L):
        gh = jnp.dot(h, whht_ref[...],
                     preferred_element_type=jnp.float32, precision=_PREC)
        gh = gh + bhh_ref[...]
        gi_t = gi[t]
        r = jax.nn.sigmoid(gi_t[:, :_D] + gh[:, :_D])
        z = jax.nn.sigmoid(gi_t[:, _D:2 * _D] + gh[:, _D:2 * _D])
        n = jnp.tanh(gi_t[:, 2 * _D:] + r * gh[:, 2 * _D:])
        hn = (1.0 - z) * n + z * h
        h = jnp.where(t < lens, hn, h)
    out_ref[...] = _normalize_rows(h)


def _gru(paths4, len3, emb_t4, rel_pad, w_ih_t, w_hh_t, b_ih2, b_hh2):
    nblk = _P // _GRU_BLK
    full = lambda i: (0, 0)
    return pl.pallas_call(
        _gru_body,
        grid=(nblk,),
        in_specs=[
            pl.BlockSpec((_L, _GRU_BLK, 1), lambda i: (0, i, 0)),
            pl.BlockSpec((1, 1, _GRU_BLK), lambda i: (i, 0, 0)),
            pl.BlockSpec((_L, 1, _GRU_BLK, _D), lambda i: (0, i, 0, 0)),
            pl.BlockSpec((_REL_PAD, _D), full),
            pl.BlockSpec((_D, 3 * _D), full),
            pl.BlockSpec((_D, 3 * _D), full),
            pl.BlockSpec((1, 3 * _D), full),
            pl.BlockSpec((1, 3 * _D), full),
        ],
        out_specs=pl.BlockSpec((_GRU_BLK, _D), lambda i: (i, 0)),
        out_shape=jax.ShapeDtypeStruct((_P, _D), jnp.float32),
    )(paths4, len3, emb_t4, rel_pad, w_ih_t, w_hh_t, b_ih2, b_hh2)


def _score_body(rows_ref, q_ref, out_ref):
    rows = rows_ref[...].reshape(_SC_BLK, _K, _D)
    q = q_ref[...]
    s = jnp.sum(rows * q[:, None, :], axis=2)
    m = jnp.max(s, axis=1)
    out_ref[...] = jnp.maximum(m, 0.0).reshape(1, 1, _SC_BLK)


def _score(rows, q):
    nblk = _B // _SC_BLK
    out = pl.pallas_call(
        _score_body,
        grid=(nblk,),
        in_specs=[
            pl.BlockSpec((_SC_BLK * _K, _D), lambda i: (i, 0)),
            pl.BlockSpec((_SC_BLK, _D), lambda i: (i, 0)),
        ],
        out_specs=pl.BlockSpec((1, 1, _SC_BLK), lambda i: (i, 0, 0)),
        out_shape=jax.ShapeDtypeStruct((nblk, 1, _SC_BLK), jnp.float32),
    )(rows, q)
    return out.reshape(_B)


def _gather(table, idx, n_rows, chunk):
    return _make_sc_gather(n_rows, chunk)(table, idx)


def kernel(path_index, batch_relation, paths, paths_time, lengths, path_r,
           path_neg_index, batch_his_r, relation_embeddings, time_table,
           W_lin, b_lin, W_ih, W_hh, b_ih, b_hh):
    f32 = jnp.float32
    tt_pad = jnp.zeros((_TT_PAD, _D), f32).at[:time_table.shape[0]].set(time_table)
    rel_pad = jnp.zeros((_REL_PAD, _D), f32).at[:_NUM_R].set(relation_embeddings)
    w_lin_t = W_lin.T
    w_ih_t = W_ih.T
    w_hh_t = W_hh.T
    b_lin2 = b_lin.reshape(1, _D)
    b_ih2 = b_ih.reshape(1, 3 * _D)
    b_hh2 = b_hh.reshape(1, 3 * _D)

    tt_lin, rel_n = _prep(tt_pad, w_lin_t, b_lin2, rel_pad)

    idx_t = paths_time.T.reshape(_NW, (_P * _L) // (_NW * 128), 128)
    emb_t = _gather(tt_lin, idx_t, _P * _L, 128)
    emb_t4 = emb_t.reshape(_L, _P // _GRU_BLK, _GRU_BLK, _D)

    paths4 = paths.T.reshape(_L, _P, 1)
    len3 = lengths.reshape(_P // _GRU_BLK, 1, _GRU_BLK).astype(jnp.int32)
    hidden_n = _gru(paths4, len3, emb_t4, rel_pad, w_ih_t, w_hh_t, b_ih2, b_hh2)

    path_tbl = jnp.concatenate([jnp.zeros((8, _D), f32), hidden_n], axis=0)

    idx_p = (jnp.concatenate([path_index.reshape(-1), path_neg_index]) + 7)
    idx_p = idx_p.reshape(_NW, 11, 96)
    rows_p = _gather(path_tbl, idx_p, _B * (_K + 1), 96)

    idx_r = jnp.concatenate([path_r, batch_relation]).reshape(_NW, 1, 64)
    rows_r = _gather(rel_n, idx_r, 2 * _B, 64)

    max_score = _score(rows_p[:_B * _K], rows_r[_B:])
    return (max_score, rows_p[_B * _K:], rows_r[:_B])
